# Initial kernel scaffold; baseline (speedup 1.0000x reference)
import jax
import jax.numpy as jnp
from jax.experimental import pallas as pl


def kernel(indices, emb_table, Wx, Wh, b):
    raise NotImplementedError("write your pallas kernel here")


# speedup vs baseline: 3.0881x; 3.0881x over previous
import functools

import jax
import jax.numpy as jnp
from jax import lax
from jax.experimental import pallas as pl
from jax.experimental.pallas import tpu as pltpu
from jax.experimental.pallas import tpu_sc as plsc

NC = 2
NS = 16
NW = NC * NS
CHUNK = 128


def _sc_gather(idx3, emb_table, n_chunks, D):
    per_w = n_chunks * CHUNK
    BT = NW * per_w
    mesh = plsc.VectorSubcoreMesh(core_axis_name="c", subcore_axis_name="s")

    @functools.partial(
        pl.kernel,
        mesh=mesh,
        compiler_params=pltpu.CompilerParams(use_tc_tiling_on_sc=False),
        out_type=jax.ShapeDtypeStruct((BT, D), jnp.float32),
        scratch_types=[
            pltpu.VMEM((n_chunks, CHUNK), jnp.int32),
            pltpu.VMEM((n_chunks, CHUNK, D), jnp.float32),
            pltpu.SemaphoreType.DMA,
            pltpu.SemaphoreType.DMA,
        ],
    )
    def gather_sc(idx_hbm, table_hbm, out_hbm, idx_v, rows_v, gsem, osem):
        wid = lax.axis_index("s") * NC + lax.axis_index("c")
        base = wid * per_w
        pltpu.sync_copy(idx_hbm.at[wid], idx_v)
        gathers = [
            pltpu.async_copy(table_hbm.at[idx_v.at[j]], rows_v.at[j], gsem)
            for j in range(n_chunks)
        ]
        for g in gathers:
            g.wait()
        outs = [
            pltpu.async_copy(
                rows_v.at[j], out_hbm.at[pl.ds(base + j * CHUNK, CHUNK)], osem
            )
            for j in range(n_chunks)
        ]
        for o in outs:
            o.wait()

    return gather_sc(idx3, emb_table)


def kernel(indices, emb_table, Wx, Wh, b):
    B, T = indices.shape
    V, D = emb_table.shape
    U = Wh.shape[0]
    G = 4 * U
    BT = B * T
    per_w = BT // NW
    n_chunks = per_w // CHUNK

    idx3 = indices.astype(jnp.int32).T.reshape(NW, n_chunks, CHUNK)
    x_tm = _sc_gather(idx3, emb_table, n_chunks, D)
    x3 = x_tm.reshape(T, B, D)

    Bt = 512
    nb = B // Bt

    def lstm_body(x_ref, wx_ref, wh_ref, b_ref, seq_ref, h_ref, c_ref, xw_ref):
        x = x_ref[...]
        xw = jnp.dot(
            x.reshape(T * Bt, D), wx_ref[...], preferred_element_type=jnp.float32
        )
        xw_ref[...] = xw.reshape(T, Bt, G) + b_ref[...]
        wh = wh_ref[...]
        h = jnp.zeros((Bt, U), jnp.float32)
        c = jnp.zeros((Bt, U), jnp.float32)
        for t in range(T):
            if t == 0:
                z = xw_ref[0]
            else:
                z = xw_ref[t] + jnp.dot(h, wh, preferred_element_type=jnp.float32)
            i = jax.nn.sigmoid(z[:, 0:U])
            f = jax.nn.sigmoid(z[:, U:2 * U])
            g = jnp.tanh(z[:, 2 * U:3 * U])
            o = jax.nn.sigmoid(z[:, 3 * U:4 * U])
            c = f * c + i * g
            h = o * jnp.tanh(c)
            seq_ref[t] = h
        h_ref[...] = h
        c_ref[...] = c

    seq_tm, h_T, c_T = pl.pallas_call(
        lstm_body,
        grid=(nb,),
        in_specs=[
            pl.BlockSpec((T, Bt, D), lambda i: (0, i, 0)),
            pl.BlockSpec((D, G), lambda i: (0, 0)),
            pl.BlockSpec((U, G), lambda i: (0, 0)),
            pl.BlockSpec((1, G), lambda i: (0, 0)),
        ],
        out_specs=[
            pl.BlockSpec((T, Bt, U), lambda i: (0, i, 0)),
            pl.BlockSpec((Bt, U), lambda i: (i, 0)),
            pl.BlockSpec((Bt, U), lambda i: (i, 0)),
        ],
        out_shape=[
            jax.ShapeDtypeStruct((T, B, U), jnp.float32),
            jax.ShapeDtypeStruct((B, U), jnp.float32),
            jax.ShapeDtypeStruct((B, U), jnp.float32),
        ],
        scratch_shapes=[pltpu.VMEM((T, Bt, G), jnp.float32)],
    )(x3, Wx, Wh, b.reshape(1, G))

    seq = seq_tm.transpose(1, 0, 2)
    return seq, h_T, c_T
